# baseline (speedup 1.0000x reference)
import dataclasses
import functools

import jax
import jax.numpy as jnp
from jax import lax
from jax.experimental import pallas as pl
from jax.experimental.pallas import tpu as pltpu
from jax.experimental.pallas import tpu_sc as plsc

N_NODES = 100000
N_EDGES = 6400000
PEN = 0.01

NC, NS, L = 2, 16, 16
NW = NC * NS
NPAD = 100352
SLICE = NPAD // NS
EDGES_PER_TILE = N_EDGES // NW
CH = 1600
NCHUNKS = EDGES_PER_TILE // CH
NTRIPLES = (NCHUNKS - 2) // 3


def _sc_edge_kernel(ei_hbm, w_hbm, traffic_hbm, out_hbm,
                    table,
                    idx0, w0, val0,
                    idx1, w1_, val1,
                    idx2, w2_, val2,
                    acc,
                    isem0, isem1, isem2, ssem0, ssem1, ssem2):
    c = lax.axis_index("c")
    s = lax.axis_index("s")
    wid = c * NS + s
    base = wid * EDGES_PER_TILE

    sets = ((idx0, w0, val0, isem0, ssem0),
            (idx1, w1_, val1, isem1, ssem1),
            (idx2, w2_, val2, isem2, ssem2))

    def fire_in(n, st):
        idx, wb, _, isem, _ = st
        e0 = base + n * CH
        pltpu.async_copy(ei_hbm.at[pl.ds(e0, CH)], idx.at[pl.ds(0, CH)], isem)
        pltpu.async_copy(ei_hbm.at[pl.ds(N_EDGES + e0, CH)],
                         idx.at[pl.ds(CH, CH)], isem)
        pltpu.async_copy(w_hbm.at[pl.ds(e0, CH)], wb, isem)

    def wait_in(st):
        idx, wb, _, isem, _ = st
        pltpu.make_async_copy(ei_hbm.at[pl.ds(0, CH)],
                              idx.at[pl.ds(0, CH)], isem).wait()
        pltpu.make_async_copy(ei_hbm.at[pl.ds(0, CH)],
                              idx.at[pl.ds(CH, CH)], isem).wait()
        pltpu.make_async_copy(w_hbm.at[pl.ds(0, CH)], wb, isem).wait()

    def compute(st):
        idx, wb, val, _, _ = st

        @pl.loop(0, CH, step=L)
        def _(i):
            si = idx[pl.ds(i, L)]
            di = idx[pl.ds(CH + i, L)]
            wv = wb[pl.ds(i, L)]
            sv = plsc.load_gather(table, [si])
            dv = plsc.load_gather(table, [di])
            t = jnp.abs(sv - dv) * (wv * PEN)
            val[pl.ds(CH + i, L)] = t
            val[pl.ds(i, L)] = -t

    def fire_scatter(st):
        idx, _, val, _, ssem = st
        pltpu.async_copy(val, acc.at[idx], ssem, add=True)

    def wait_scatter(st):
        idx, _, val, _, ssem = st
        pltpu.make_async_copy(val, acc.at[idx], ssem).wait()

    pltpu.sync_copy(traffic_hbm, table)

    @pl.loop(0, 2 * CH, step=L)
    def _(i):
        val0[pl.ds(i, L)] = jnp.zeros((L,), jnp.float32)

    pltpu.sync_copy(val0, acc.at[pl.ds(s * SLICE, 2 * CH)])
    rem = SLICE - 2 * CH
    pltpu.sync_copy(val0.at[pl.ds(0, rem)],
                    acc.at[pl.ds(s * SLICE + 2 * CH, rem)])
    plsc.subcore_barrier()

    fire_in(0, sets[0])

    def phase(n, j, k, guarded):
        p = j % 3
        pn = (j + 1) % 3
        if guarded:
            @pl.when(k > 0)
            def _():
                wait_scatter(sets[pn])
        else:
            wait_scatter(sets[pn])

        fire_in(n + 1, sets[pn])
        wait_in(sets[p])
        compute(sets[p])
        fire_scatter(sets[p])

    @pl.loop(0, NTRIPLES)
    def _(k):
        phase(3 * k, 0, k, True)
        phase(3 * k + 1, 1, k, True)
        phase(3 * k + 2, 2, k, False)

    n0 = 3 * NTRIPLES

    def phase_static(n):
        p = n % 3
        pn = (n + 1) % 3
        wait_scatter(sets[pn])
        if n + 1 <= NCHUNKS - 1:
            fire_in(n + 1, sets[pn])
        wait_in(sets[p])
        compute(sets[p])
        fire_scatter(sets[p])

    phase_static(n0)
    phase_static(n0 + 1)
    wait_scatter(sets[n0 % 3])
    wait_scatter(sets[(n0 + 1) % 3])

    plsc.subcore_barrier()
    pltpu.sync_copy(acc.at[pl.ds(s * SLICE, 2 * CH)], val0)
    pltpu.sync_copy(val0, out_hbm.at[pl.ds(c * NPAD + s * SLICE, 2 * CH)])
    pltpu.sync_copy(acc.at[pl.ds(s * SLICE + 2 * CH, rem)],
                    val0.at[pl.ds(0, rem)])
    pltpu.sync_copy(val0.at[pl.ds(0, rem)],
                    out_hbm.at[pl.ds(c * NPAD + s * SLICE + 2 * CH, rem)])


def _tc_combine_kernel(acc_ref, t_ref, y_ref, c_ref, new_ref, eff_ref):
    new = t_ref[...] + acc_ref[0] + acc_ref[1]
    new_ref[...] = new
    eff = jnp.sum(y_ref[...] * new) - jnp.sum(c_ref[...])
    eff_ref[...] = eff.reshape(1, 1)


def kernel(edge_index, edge_weight, nodes_yield_rate, nodes_traffic, nodes_cost):
    ei_flat = edge_index.astype(jnp.int32).reshape(2 * N_EDGES)
    w1 = edge_weight.astype(jnp.float32)

    mesh = plsc.VectorSubcoreMesh(core_axis_name="c", subcore_axis_name="s")
    cp = pltpu.CompilerParams()
    if "needs_layout_passes" in pltpu.CompilerParams.__dataclass_fields__:
        cp = dataclasses.replace(cp, needs_layout_passes=False)
    sc_call = functools.partial(
        pl.kernel,
        compiler_params=cp,
        out_type=jax.ShapeDtypeStruct((NC * NPAD,), jnp.float32),
        mesh=mesh,
        scratch_types=[
            pltpu.VMEM((N_NODES,), jnp.float32),
            pltpu.VMEM((2 * CH,), jnp.int32),
            pltpu.VMEM((CH,), jnp.float32),
            pltpu.VMEM((2 * CH,), jnp.float32),
            pltpu.VMEM((2 * CH,), jnp.int32),
            pltpu.VMEM((CH,), jnp.float32),
            pltpu.VMEM((2 * CH,), jnp.float32),
            pltpu.VMEM((2 * CH,), jnp.int32),
            pltpu.VMEM((CH,), jnp.float32),
            pltpu.VMEM((2 * CH,), jnp.float32),
            pltpu.VMEM_SHARED((NPAD,), jnp.float32),
            pltpu.SemaphoreType.DMA,
            pltpu.SemaphoreType.DMA,
            pltpu.SemaphoreType.DMA,
            pltpu.SemaphoreType.DMA,
            pltpu.SemaphoreType.DMA,
            pltpu.SemaphoreType.DMA,
        ],
    )(_sc_edge_kernel)
    accs = sc_call(ei_flat, w1, nodes_traffic)

    npad = NPAD - N_NODES
    t2 = jnp.pad(nodes_traffic, (0, npad)).reshape(NPAD // 128, 128)
    y2 = jnp.pad(nodes_yield_rate, (0, npad)).reshape(NPAD // 128, 128)
    c2 = jnp.pad(nodes_cost, (0, npad)).reshape(NPAD // 128, 128)
    acc3 = accs.reshape(NC, NPAD // 128, 128)

    new2, eff = pl.pallas_call(
        _tc_combine_kernel,
        out_shape=[
            jax.ShapeDtypeStruct((NPAD // 128, 128), jnp.float32),
            jax.ShapeDtypeStruct((1, 1), jnp.float32),
        ],
    )(acc3, t2, y2, c2)

    new_traffic = new2.reshape(NPAD)[:N_NODES]
    return (new_traffic, eff[0, 0])
